# initial kernel scaffold (unmeasured)
import jax
import jax.numpy as jnp
from jax import lax
from jax.experimental import pallas as pl
from jax.experimental.pallas import tpu as pltpu

N_DEV = 8
M, N = 4096, 2048
CH = M // N_DEV
NSLOTS = 4


def kernel(x, w_mat, scale_x, scale_w):
    def body(x_ref, w_ref, sx_ref, sw_ref, out_ref,
             comm_ref, rs_send, rs_recv, ag_send, ag_recv, credit_sem):
        my = lax.axis_index("i")
        left = lax.rem(my - 1 + N_DEV, N_DEV)
        right = lax.rem(my + 1, N_DEV)

        out_ref[...] = lax.dot_general(
            x_ref[...], w_ref[...], (((1,), (0,)), ((), ())),
            preferred_element_type=jnp.float32)

        barrier = pltpu.get_barrier_semaphore()
        for nbr in (left, right):
            pl.semaphore_signal(barrier, inc=1, device_id=(nbr,),
                                device_id_type=pl.DeviceIdType.MESH)
        pl.semaphore_wait(barrier, 2)

        for s in range(N_DEV - 1):
            c = lax.rem(my - s + N_DEV, N_DEV)
            slot = s % NSLOTS
            if s >= NSLOTS:
                pl.semaphore_wait(credit_sem, 1)
            rdma = pltpu.make_async_remote_copy(
                src_ref=out_ref.at[pl.ds(c * CH, CH), :],
                dst_ref=comm_ref.at[slot],
                send_sem=rs_send.at[s],
                recv_sem=rs_recv.at[s],
                device_id=(right,),
                device_id_type=pl.DeviceIdType.MESH)
            rdma.start()
            rdma.wait()
            rc = lax.rem(my - s - 1 + N_DEV, N_DEV)
            out_ref[pl.ds(rc * CH, CH), :] += comm_ref[slot]
            if s < (N_DEV - 1) - NSLOTS:
                pl.semaphore_signal(credit_sem, inc=1, device_id=(left,),
                                    device_id_type=pl.DeviceIdType.MESH)

        scale = sx_ref[0] * sw_ref[0]
        own = lax.rem(my + 1, N_DEV)
        blk = out_ref[pl.ds(own * CH, CH), :]
        out_ref[pl.ds(own * CH, CH), :] = jnp.maximum(blk * scale, 0.0)

        for t in range(N_DEV - 1):
            g = lax.rem(my + 1 - t + N_DEV, N_DEV)
            rdma = pltpu.make_async_remote_copy(
                src_ref=out_ref.at[pl.ds(g * CH, CH), :],
                dst_ref=out_ref.at[pl.ds(g * CH, CH), :],
                send_sem=ag_send.at[t],
                recv_sem=ag_recv.at[t],
                device_id=(right,),
                device_id_type=pl.DeviceIdType.MESH)
            rdma.start()
            rdma.wait()

    return pl.pallas_call(
        body,
        out_shape=jax.ShapeDtypeStruct((M, N), jnp.float32),
        in_specs=[
            pl.BlockSpec(memory_space=pltpu.VMEM),
            pl.BlockSpec(memory_space=pltpu.VMEM),
            pl.BlockSpec(memory_space=pltpu.SMEM),
            pl.BlockSpec(memory_space=pltpu.SMEM),
        ],
        out_specs=pl.BlockSpec(memory_space=pltpu.VMEM),
        scratch_shapes=[
            pltpu.VMEM((NSLOTS, CH, N), jnp.float32),
            pltpu.SemaphoreType.DMA((N_DEV - 1,)),
            pltpu.SemaphoreType.DMA((N_DEV - 1,)),
            pltpu.SemaphoreType.DMA((N_DEV - 1,)),
            pltpu.SemaphoreType.DMA((N_DEV - 1,)),
            pltpu.SemaphoreType.REGULAR,
        ],
        compiler_params=pltpu.CompilerParams(collective_id=0),
    )(x, w_mat, scale_x, scale_w)


# baseline (device time: 709429 ns/iter reference)
import jax
import jax.numpy as jnp
from jax import lax
from jax.experimental import pallas as pl
from jax.experimental.pallas import tpu as pltpu

N_DEV = 8
M, N = 4096, 2048
CH = M // N_DEV
NSLOTS = 2


def kernel(x, w_mat, scale_x, scale_w):
    def body(x_ref, w_ref, sx_ref, sw_ref, out_ref,
             comm_ref, rs_send, rs_recv, ag_send, ag_recv, credit_sem):
        my = lax.axis_index("i")
        left = lax.rem(my - 1 + N_DEV, N_DEV)
        right = lax.rem(my + 1, N_DEV)

        out_ref[...] = lax.dot_general(
            x_ref[...].astype(jnp.bfloat16), w_ref[...].astype(jnp.bfloat16),
            (((1,), (0,)), ((), ())),
            preferred_element_type=jnp.float32)

        barrier = pltpu.get_barrier_semaphore()
        for nbr in (left, right):
            pl.semaphore_signal(barrier, inc=1, device_id=(nbr,),
                                device_id_type=pl.DeviceIdType.MESH)
        pl.semaphore_wait(barrier, 2)

        for s in range(N_DEV - 1):
            c = lax.rem(my - s + N_DEV, N_DEV)
            slot = s % NSLOTS
            if s >= NSLOTS:
                pl.semaphore_wait(credit_sem, 1)
            rdma = pltpu.make_async_remote_copy(
                src_ref=out_ref.at[pl.ds(c * CH, CH), :],
                dst_ref=comm_ref.at[slot],
                send_sem=rs_send.at[s],
                recv_sem=rs_recv.at[s],
                device_id=(right,),
                device_id_type=pl.DeviceIdType.MESH)
            rdma.start()
            rdma.wait()
            rc = lax.rem(my - s - 1 + N_DEV, N_DEV)
            out_ref[pl.ds(rc * CH, CH), :] += comm_ref[slot]
            if s < (N_DEV - 1) - NSLOTS:
                pl.semaphore_signal(credit_sem, inc=1, device_id=(left,),
                                    device_id_type=pl.DeviceIdType.MESH)

        scale = sx_ref[0] * sw_ref[0]
        own = lax.rem(my + 1, N_DEV)
        blk = out_ref[pl.ds(own * CH, CH), :]
        out_ref[pl.ds(own * CH, CH), :] = jnp.maximum(blk * scale, 0.0)

        for t in range(N_DEV - 1):
            g = lax.rem(my + 1 - t + N_DEV, N_DEV)
            rdma = pltpu.make_async_remote_copy(
                src_ref=out_ref.at[pl.ds(g * CH, CH), :],
                dst_ref=out_ref.at[pl.ds(g * CH, CH), :],
                send_sem=ag_send.at[t],
                recv_sem=ag_recv.at[t],
                device_id=(right,),
                device_id_type=pl.DeviceIdType.MESH)
            rdma.start()
            rdma.wait()

    return pl.pallas_call(
        body,
        out_shape=jax.ShapeDtypeStruct((M, N), jnp.float32),
        in_specs=[
            pl.BlockSpec(memory_space=pltpu.VMEM),
            pl.BlockSpec(memory_space=pltpu.VMEM),
            pl.BlockSpec(memory_space=pltpu.SMEM),
            pl.BlockSpec(memory_space=pltpu.SMEM),
        ],
        out_specs=pl.BlockSpec(memory_space=pltpu.VMEM),
        scratch_shapes=[
            pltpu.VMEM((NSLOTS, CH, N), jnp.float32),
            pltpu.SemaphoreType.DMA((N_DEV - 1,)),
            pltpu.SemaphoreType.DMA((N_DEV - 1,)),
            pltpu.SemaphoreType.DMA((N_DEV - 1,)),
            pltpu.SemaphoreType.DMA((N_DEV - 1,)),
            pltpu.SemaphoreType.REGULAR,
        ],
        compiler_params=pltpu.CompilerParams(
            collective_id=0, vmem_limit_bytes=64 * 1024 * 1024),
    )(x, w_mat, scale_x, scale_w)


# device time: 253425 ns/iter; 2.7994x vs baseline; 2.7994x over previous
import jax
import jax.numpy as jnp
from jax import lax
from jax.experimental import pallas as pl
from jax.experimental.pallas import tpu as pltpu

N_DEV = 8
M, N = 4096, 2048
CH = M // N_DEV
HC = N // 2
NSLOTS = 2
NSTEPS = 2 * (N_DEV - 1)


def kernel(x, w_mat, scale_x, scale_w):
    def body(x_ref, w_ref, sx_ref, sw_ref, out_ref,
             stage0, comm0, stage1, comm1,
             send0, recv0, send1, recv1, credit0, credit1):
        my = lax.axis_index("i")
        left = lax.rem(my + N_DEV - 1, N_DEV)
        right = lax.rem(my + 1, N_DEV)

        def rows(c):
            return pl.ds(c * CH, CH)

        cols0 = pl.ds(0, HC)
        cols1 = pl.ds(HC, HC)

        out_ref[...] = lax.dot_general(
            x_ref[...].astype(jnp.bfloat16), w_ref[...].astype(jnp.bfloat16),
            (((1,), (0,)), ((), ())),
            preferred_element_type=jnp.float32)

        barrier = pltpu.get_barrier_semaphore()
        for nbr in (left, right):
            pl.semaphore_signal(barrier, inc=1, device_id=(nbr,),
                                device_id_type=pl.DeviceIdType.MESH)
        pl.semaphore_wait(barrier, 2)

        def copy(src, dst, ssem, rsem, dev):
            return pltpu.make_async_remote_copy(
                src_ref=src, dst_ref=dst, send_sem=ssem, recv_sem=rsem,
                device_id=(dev,), device_id_type=pl.DeviceIdType.MESH)

        for s in range(N_DEV - 1):
            slot = s % NSLOTS
            c0 = lax.rem(my - s + N_DEV, N_DEV)
            c1 = lax.rem(my + s, N_DEV)
            if s >= NSLOTS:
                pl.semaphore_wait(credit0, 1)
                pl.semaphore_wait(credit1, 1)
            stage0[slot] = out_ref[rows(c0), cols0].astype(jnp.bfloat16)
            stage1[slot] = out_ref[rows(c1), cols1].astype(jnp.bfloat16)
            r0 = copy(stage0.at[slot], comm0.at[slot],
                      send0.at[s], recv0.at[s], right)
            r1 = copy(stage1.at[slot], comm1.at[slot],
                      send1.at[s], recv1.at[s], left)
            r0.start()
            r1.start()
            r0.wait()
            r1.wait()
            rc0 = lax.rem(my - s + N_DEV - 1, N_DEV)
            rc1 = lax.rem(my + s + 1, N_DEV)
            out_ref[rows(rc0), cols0] += comm0[slot].astype(jnp.float32)
            out_ref[rows(rc1), cols1] += comm1[slot].astype(jnp.float32)
            pl.semaphore_signal(credit0, inc=1, device_id=(left,),
                                device_id_type=pl.DeviceIdType.MESH)
            pl.semaphore_signal(credit1, inc=1, device_id=(right,),
                                device_id_type=pl.DeviceIdType.MESH)

        scale = sx_ref[0] * sw_ref[0]
        own0, own1 = right, left
        blk0 = jnp.maximum(out_ref[rows(own0), cols0] * scale, 0.0)
        blk1 = jnp.maximum(out_ref[rows(own1), cols1] * scale, 0.0)
        out_ref[rows(own0), cols0] = blk0
        out_ref[rows(own1), cols1] = blk1
        first_slot = (N_DEV - 1) % NSLOTS
        stage0[first_slot] = blk0.astype(jnp.bfloat16)
        stage1[first_slot] = blk1.astype(jnp.bfloat16)

        for t in range(N_DEV - 1):
            u = (N_DEV - 1) + t
            slot = u % NSLOTS
            pl.semaphore_wait(credit0, 1)
            pl.semaphore_wait(credit1, 1)
            if t == 0:
                src0, src1 = stage0.at[first_slot], stage1.at[first_slot]
            else:
                src0 = comm0.at[(u - 1) % NSLOTS]
                src1 = comm1.at[(u - 1) % NSLOTS]
            r0 = copy(src0, comm0.at[slot], send0.at[u], recv0.at[u], right)
            r1 = copy(src1, comm1.at[slot], send1.at[u], recv1.at[u], left)
            r0.start()
            r1.start()
            r0.wait()
            r1.wait()
            g0 = lax.rem(my - t + N_DEV, N_DEV)
            g1 = lax.rem(my + t, N_DEV)
            out_ref[rows(g0), cols0] = comm0[slot].astype(jnp.float32)
            out_ref[rows(g1), cols1] = comm1[slot].astype(jnp.float32)
            if 1 <= t <= 5:
                pl.semaphore_signal(credit0, inc=1, device_id=(left,),
                                    device_id_type=pl.DeviceIdType.MESH)
                pl.semaphore_signal(credit1, inc=1, device_id=(right,),
                                    device_id_type=pl.DeviceIdType.MESH)

    return pl.pallas_call(
        body,
        out_shape=jax.ShapeDtypeStruct((M, N), jnp.float32),
        in_specs=[
            pl.BlockSpec(memory_space=pltpu.VMEM),
            pl.BlockSpec(memory_space=pltpu.VMEM),
            pl.BlockSpec(memory_space=pltpu.SMEM),
            pl.BlockSpec(memory_space=pltpu.SMEM),
        ],
        out_specs=pl.BlockSpec(memory_space=pltpu.VMEM),
        scratch_shapes=[
            pltpu.VMEM((NSLOTS, CH, HC), jnp.bfloat16),
            pltpu.VMEM((NSLOTS, CH, HC), jnp.bfloat16),
            pltpu.VMEM((NSLOTS, CH, HC), jnp.bfloat16),
            pltpu.VMEM((NSLOTS, CH, HC), jnp.bfloat16),
            pltpu.SemaphoreType.DMA((NSTEPS,)),
            pltpu.SemaphoreType.DMA((NSTEPS,)),
            pltpu.SemaphoreType.DMA((NSTEPS,)),
            pltpu.SemaphoreType.DMA((NSTEPS,)),
            pltpu.SemaphoreType.REGULAR,
            pltpu.SemaphoreType.REGULAR,
        ],
        compiler_params=pltpu.CompilerParams(
            collective_id=0, vmem_limit_bytes=64 * 1024 * 1024),
    )(x, w_mat, scale_x, scale_w)


# device time: 245836 ns/iter; 2.8858x vs baseline; 1.0309x over previous
import jax
import jax.numpy as jnp
from jax import lax
from jax.experimental import pallas as pl
from jax.experimental.pallas import tpu as pltpu

N_DEV = 8
M, N = 4096, 2048
CH = M // N_DEV
NRING = 4
QC = N // NRING
NSLOTS = 2
NSTEPS = 2 * (N_DEV - 1)


def kernel(x, w_mat, scale_x, scale_w):
    def body(x_ref, w_ref, sx_ref, sw_ref, out_ref,
             stage, comm, sends, recvs, credits):
        my = lax.axis_index("i")
        left = lax.rem(my + N_DEV - 1, N_DEV)
        right = lax.rem(my + 1, N_DEV)

        def rows(c):
            return pl.ds(c * CH, CH)

        rings = []
        for r in range(NRING):
            cw = r < NRING // 2
            rings.append(dict(
                sg=-1 if cw else 1,
                dst=right if cw else left,
                credit_to=left if cw else right,
                co=pl.ds(r * QC, QC),
            ))

        out_ref[...] = lax.dot_general(
            x_ref[...].astype(jnp.bfloat16), w_ref[...].astype(jnp.bfloat16),
            (((1,), (0,)), ((), ())),
            preferred_element_type=jnp.float32)

        barrier = pltpu.get_barrier_semaphore()
        for nbr in (left, right):
            pl.semaphore_signal(barrier, inc=1, device_id=(nbr,),
                                device_id_type=pl.DeviceIdType.MESH)
        pl.semaphore_wait(barrier, 2)

        def copy(src, dst, ssem, rsem, dev):
            return pltpu.make_async_remote_copy(
                src_ref=src, dst_ref=dst, send_sem=ssem, recv_sem=rsem,
                device_id=(dev,), device_id_type=pl.DeviceIdType.MESH)

        for s in range(N_DEV - 1):
            slot = s % NSLOTS
            rdmas = []
            for r, cfg in enumerate(rings):
                if s >= NSLOTS:
                    pl.semaphore_wait(credits.at[r], 1)
                c = lax.rem(my + cfg["sg"] * s + 8 * N_DEV, N_DEV)
                stage[r, slot] = out_ref[rows(c), cfg["co"]].astype(jnp.bfloat16)
                rd = copy(stage.at[r, slot], comm.at[r, slot],
                          sends.at[r, s], recvs.at[r, s], cfg["dst"])
                rd.start()
                rdmas.append(rd)
            for r, cfg in enumerate(rings):
                rdmas[r].wait()
                rc = lax.rem(my + cfg["sg"] * (s + 1) + 8 * N_DEV, N_DEV)
                out_ref[rows(rc), cfg["co"]] += comm[r, slot].astype(jnp.float32)
                pl.semaphore_signal(credits.at[r], inc=1,
                                    device_id=(cfg["credit_to"],),
                                    device_id_type=pl.DeviceIdType.MESH)

        scale = sx_ref[0] * sw_ref[0]
        first_slot = (N_DEV - 1) % NSLOTS
        for r, cfg in enumerate(rings):
            own = lax.rem(my - cfg["sg"] + N_DEV, N_DEV)
            blk = jnp.maximum(out_ref[rows(own), cfg["co"]] * scale, 0.0)
            out_ref[rows(own), cfg["co"]] = blk
            stage[r, first_slot] = blk.astype(jnp.bfloat16)

        for t in range(N_DEV - 1):
            u = (N_DEV - 1) + t
            slot = u % NSLOTS
            rdmas = []
            for r, cfg in enumerate(rings):
                pl.semaphore_wait(credits.at[r], 1)
                if t == 0:
                    src = stage.at[r, first_slot]
                else:
                    src = comm.at[r, (u - 1) % NSLOTS]
                rd = copy(src, comm.at[r, slot],
                          sends.at[r, u], recvs.at[r, u], cfg["dst"])
                rd.start()
                rdmas.append(rd)
            for r, cfg in enumerate(rings):
                rdmas[r].wait()
                g = lax.rem(my + cfg["sg"] * t + 8 * N_DEV, N_DEV)
                out_ref[rows(g), cfg["co"]] = comm[r, slot].astype(jnp.float32)
                if 1 <= t <= 5:
                    pl.semaphore_signal(credits.at[r], inc=1,
                                        device_id=(cfg["credit_to"],),
                                        device_id_type=pl.DeviceIdType.MESH)

    return pl.pallas_call(
        body,
        out_shape=jax.ShapeDtypeStruct((M, N), jnp.float32),
        in_specs=[
            pl.BlockSpec(memory_space=pltpu.VMEM),
            pl.BlockSpec(memory_space=pltpu.VMEM),
            pl.BlockSpec(memory_space=pltpu.SMEM),
            pl.BlockSpec(memory_space=pltpu.SMEM),
        ],
        out_specs=pl.BlockSpec(memory_space=pltpu.VMEM),
        scratch_shapes=[
            pltpu.VMEM((NRING, NSLOTS, CH, QC), jnp.bfloat16),
            pltpu.VMEM((NRING, NSLOTS, CH, QC), jnp.bfloat16),
            pltpu.SemaphoreType.DMA((NRING, NSTEPS)),
            pltpu.SemaphoreType.DMA((NRING, NSTEPS)),
            pltpu.SemaphoreType.REGULAR((NRING,)),
        ],
        compiler_params=pltpu.CompilerParams(
            collective_id=0, vmem_limit_bytes=64 * 1024 * 1024),
    )(x, w_mat, scale_x, scale_w)


# device time: 237820 ns/iter; 2.9831x vs baseline; 1.0337x over previous
import jax
import jax.numpy as jnp
from jax import lax
from jax.experimental import pallas as pl
from jax.experimental.pallas import tpu as pltpu

N_DEV = 8
M, N = 4096, 2048
CH = M // N_DEV
NRING = 4
QC = N // NRING
NSLOTS = 2
NSTEPS = 2 * (N_DEV - 1)


def kernel(x, w_mat, scale_x, scale_w):
    def body(x_ref, w_ref, sx_ref, sw_ref, out_ref,
             w_bf, stage, comm, sends, recvs, credits):
        my = lax.axis_index("i")
        left = lax.rem(my + N_DEV - 1, N_DEV)
        right = lax.rem(my + 1, N_DEV)

        def rows(c):
            return pl.ds(c * CH, CH)

        def ch(k):
            return lax.rem(my + k + 4 * N_DEV, N_DEV)

        rings = []
        for r in range(NRING):
            cw = r < NRING // 2
            rings.append(dict(
                sg=-1 if cw else 1,
                dst=right if cw else left,
                credit_to=left if cw else right,
                co=pl.ds(r * QC, QC),
            ))

        w_bf[...] = w_ref[...].astype(jnp.bfloat16)

        def gemm(k):
            c = ch(k)
            out_ref[rows(c), :] = lax.dot_general(
                x_ref[rows(c), :].astype(jnp.bfloat16), w_bf[...],
                (((1,), (0,)), ((), ())),
                preferred_element_type=jnp.float32)

        gemm(0)

        barrier = pltpu.get_barrier_semaphore()
        for nbr in (left, right):
            pl.semaphore_signal(barrier, inc=1, device_id=(nbr,),
                                device_id_type=pl.DeviceIdType.MESH)
        pl.semaphore_wait(barrier, 2)

        def copy(src, dst, ssem, rsem, dev):
            return pltpu.make_async_remote_copy(
                src_ref=src, dst_ref=dst, send_sem=ssem, recv_sem=rsem,
                device_id=(dev,), device_id_type=pl.DeviceIdType.MESH)

        for s in range(N_DEV - 1):
            slot = s % NSLOTS
            rdmas = []
            for r, cfg in enumerate(rings):
                if s >= NSLOTS:
                    pl.semaphore_wait(credits.at[r], 1)
                c = lax.rem(my + cfg["sg"] * s + 8 * N_DEV, N_DEV)
                stage[r, slot] = out_ref[rows(c), cfg["co"]].astype(jnp.bfloat16)
                rd = copy(stage.at[r, slot], comm.at[r, slot],
                          sends.at[r, s], recvs.at[r, s], cfg["dst"])
                rd.start()
                rdmas.append(rd)
            if s == 0:
                gemm(-1)
                gemm(1)
            elif s == 1:
                gemm(-2)
                gemm(2)
            elif s == 2:
                gemm(-3)
                gemm(3)
            elif s == 3:
                gemm(4)
            for r, cfg in enumerate(rings):
                rdmas[r].wait()
                rc = lax.rem(my + cfg["sg"] * (s + 1) + 8 * N_DEV, N_DEV)
                out_ref[rows(rc), cfg["co"]] += comm[r, slot].astype(jnp.float32)
                pl.semaphore_signal(credits.at[r], inc=1,
                                    device_id=(cfg["credit_to"],),
                                    device_id_type=pl.DeviceIdType.MESH)

        scale = sx_ref[0] * sw_ref[0]
        first_slot = (N_DEV - 1) % NSLOTS
        for r, cfg in enumerate(rings):
            own = lax.rem(my - cfg["sg"] + N_DEV, N_DEV)
            blk = jnp.maximum(out_ref[rows(own), cfg["co"]] * scale, 0.0)
            out_ref[rows(own), cfg["co"]] = blk
            stage[r, first_slot] = blk.astype(jnp.bfloat16)

        for t in range(N_DEV - 1):
            u = (N_DEV - 1) + t
            slot = u % NSLOTS
            rdmas = []
            for r, cfg in enumerate(rings):
                pl.semaphore_wait(credits.at[r], 1)
                if t == 0:
                    src = stage.at[r, first_slot]
                else:
                    src = comm.at[r, (u - 1) % NSLOTS]
                rd = copy(src, comm.at[r, slot],
                          sends.at[r, u], recvs.at[r, u], cfg["dst"])
                rd.start()
                rdmas.append(rd)
            for r, cfg in enumerate(rings):
                rdmas[r].wait()
                g = lax.rem(my + cfg["sg"] * t + 8 * N_DEV, N_DEV)
                out_ref[rows(g), cfg["co"]] = comm[r, slot].astype(jnp.float32)
                if 1 <= t <= 5:
                    pl.semaphore_signal(credits.at[r], inc=1,
                                        device_id=(cfg["credit_to"],),
                                        device_id_type=pl.DeviceIdType.MESH)

    return pl.pallas_call(
        body,
        out_shape=jax.ShapeDtypeStruct((M, N), jnp.float32),
        in_specs=[
            pl.BlockSpec(memory_space=pltpu.VMEM),
            pl.BlockSpec(memory_space=pltpu.VMEM),
            pl.BlockSpec(memory_space=pltpu.SMEM),
            pl.BlockSpec(memory_space=pltpu.SMEM),
        ],
        out_specs=pl.BlockSpec(memory_space=pltpu.VMEM),
        scratch_shapes=[
            pltpu.VMEM((512, N), jnp.bfloat16),
            pltpu.VMEM((NRING, NSLOTS, CH, QC), jnp.bfloat16),
            pltpu.VMEM((NRING, NSLOTS, CH, QC), jnp.bfloat16),
            pltpu.SemaphoreType.DMA((NRING, NSTEPS)),
            pltpu.SemaphoreType.DMA((NRING, NSTEPS)),
            pltpu.SemaphoreType.REGULAR((NRING,)),
        ],
        compiler_params=pltpu.CompilerParams(
            collective_id=0, vmem_limit_bytes=64 * 1024 * 1024),
    )(x, w_mat, scale_x, scale_w)


# device time: 204199 ns/iter; 3.4742x vs baseline; 1.1646x over previous
import jax
import jax.numpy as jnp
from jax import lax
from jax.experimental import pallas as pl
from jax.experimental.pallas import tpu as pltpu

N_DEV = 8
M, N = 4096, 2048
CH = M // N_DEV
NRING = 4
QC = N // NRING
NSLOTS = 2
NSTEPS = 2 * (N_DEV - 1)


def kernel(x, w_mat, scale_x, scale_w):
    def body(x_ref, w_ref, sx_ref, sw_ref, out_ref,
             w_bf, stage, comm, sends, recvs, credits):
        my = lax.axis_index("i")
        left = lax.rem(my + N_DEV - 1, N_DEV)
        right = lax.rem(my + 1, N_DEV)

        def rows(c):
            return pl.ds(c * CH, CH)

        def ch(k):
            return lax.rem(my + k + 4 * N_DEV, N_DEV)

        rings = []
        for q, cwq in ((0, True), (2, False), (1, True), (3, False)):
            rings.append(dict(
                sg=-1 if cwq else 1,
                dst=right if cwq else left,
                credit_to=left if cwq else right,
                co=pl.ds(q * QC, QC)))

        w_bf[...] = w_ref[...].astype(jnp.bfloat16)

        def gemm(k):
            c = ch(k)
            out_ref[rows(c), :] = lax.dot_general(
                x_ref[rows(c), :].astype(jnp.bfloat16), w_bf[...],
                (((1,), (0,)), ((), ())),
                preferred_element_type=jnp.float32)

        gemm(0)

        barrier = pltpu.get_barrier_semaphore()
        for nbr in (left, right):
            pl.semaphore_signal(barrier, inc=1, device_id=(nbr,),
                                device_id_type=pl.DeviceIdType.MESH)
        pl.semaphore_wait(barrier, 2)

        def copy(src, dst, ssem, rsem, dev):
            return pltpu.make_async_remote_copy(
                src_ref=src, dst_ref=dst, send_sem=ssem, recv_sem=rsem,
                device_id=(dev,), device_id_type=pl.DeviceIdType.MESH)

        def hop_src(r, u):
            if u < N_DEV - 1:
                return stage.at[r, u % NSLOTS]
            if u == N_DEV - 1:
                return stage.at[r, (N_DEV - 1) % NSLOTS]
            return comm.at[r, (u - 1) % NSLOTS]

        def start(r, u):
            copy(hop_src(r, u), comm.at[r, u % NSLOTS],
                 sends.at[r, u], recvs.at[r, u], rings[r]["dst"]).start()

        def wait_hop(r, u):
            copy(hop_src(r, u), comm.at[r, u % NSLOTS],
                 sends.at[r, u], recvs.at[r, u], rings[r]["dst"]).wait()

        def signal_credit(r):
            pl.semaphore_signal(credits.at[r], inc=1,
                                device_id=(rings[r]["credit_to"],),
                                device_id_type=pl.DeviceIdType.MESH)

        for r, cfg in enumerate(rings):
            stage[r, 0] = out_ref[rows(ch(0)), cfg["co"]].astype(jnp.bfloat16)
            start(r, 0)
        gemm(-1)
        gemm(1)

        for s in range(N_DEV - 1):
            slot = s % NSLOTS
            for r, cfg in enumerate(rings):
                wait_hop(r, s)
                rc = ch(cfg["sg"] * (s + 1))
                out_ref[rows(rc), cfg["co"]] += comm[r, slot].astype(jnp.float32)
                signal_credit(r)
                if s < N_DEV - 2:
                    if s + 1 >= NSLOTS:
                        pl.semaphore_wait(credits.at[r], 1)
                    stage[r, (s + 1) % NSLOTS] = out_ref[
                        rows(rc), cfg["co"]].astype(jnp.bfloat16)
                    start(r, s + 1)
                else:
                    scale = sx_ref[0] * sw_ref[0]
                    blk = jnp.maximum(out_ref[rows(rc), cfg["co"]] * scale, 0.0)
                    out_ref[rows(rc), cfg["co"]] = blk
                    pl.semaphore_wait(credits.at[r], 1)
                    stage[r, (N_DEV - 1) % NSLOTS] = blk.astype(jnp.bfloat16)
                    start(r, N_DEV - 1)
            if s == 0:
                gemm(-2)
                gemm(2)
            elif s == 1:
                gemm(-3)
                gemm(3)
            elif s == 2:
                gemm(4)

        for u in range(N_DEV - 1, 2 * N_DEV - 3):
            t = u - (N_DEV - 1)
            slot = u % NSLOTS
            for r, cfg in enumerate(rings):
                wait_hop(r, u)
                if u >= N_DEV:
                    signal_credit(r)
                pl.semaphore_wait(credits.at[r], 1)
                start(r, u + 1)
                g = ch(cfg["sg"] * t)
                out_ref[rows(g), cfg["co"]] = comm[r, slot].astype(jnp.float32)

        u_last = 2 * N_DEV - 3
        for r, cfg in enumerate(rings):
            wait_hop(r, u_last)
            g = ch(cfg["sg"] * (N_DEV - 2))
            out_ref[rows(g), cfg["co"]] = comm[r, u_last % NSLOTS].astype(
                jnp.float32)

    return pl.pallas_call(
        body,
        out_shape=jax.ShapeDtypeStruct((M, N), jnp.float32),
        in_specs=[
            pl.BlockSpec(memory_space=pltpu.VMEM),
            pl.BlockSpec(memory_space=pltpu.VMEM),
            pl.BlockSpec(memory_space=pltpu.SMEM),
            pl.BlockSpec(memory_space=pltpu.SMEM),
        ],
        out_specs=pl.BlockSpec(memory_space=pltpu.VMEM),
        scratch_shapes=[
            pltpu.VMEM((512, N), jnp.bfloat16),
            pltpu.VMEM((NRING, NSLOTS, CH, QC), jnp.bfloat16),
            pltpu.VMEM((NRING, NSLOTS, CH, QC), jnp.bfloat16),
            pltpu.SemaphoreType.DMA((NRING, NSTEPS)),
            pltpu.SemaphoreType.DMA((NRING, NSTEPS)),
            pltpu.SemaphoreType.REGULAR((NRING,)),
        ],
        compiler_params=pltpu.CompilerParams(
            collective_id=0, vmem_limit_bytes=64 * 1024 * 1024),
    )(x, w_mat, scale_x, scale_w)
